# Initial kernel scaffold; baseline (speedup 1.0000x reference)
import jax
import jax.numpy as jnp
from jax.experimental import pallas as pl


def kernel(x, edge_index, W1, a_src1, a_dst1, W2, a_src2, a_dst2):
    raise NotImplementedError("write your pallas kernel here")


# speedup vs baseline: 4.8669x; 4.8669x over previous
import functools

import jax
import jax.numpy as jnp
from jax import lax
from jax.experimental import pallas as pl
from jax.experimental.pallas import tpu as pltpu
from jax.experimental.pallas import tpu_sc as plsc

N = 10000
E = 320000
F_IN = 128
HID = 128
OUT = 64
ALPHA = 0.1
K = 10

NC = 2
NS = 16
NW = NC * NS
CH = 128
RPT = (-(-E // (NW * CH)) + 7) // 8 * 8
RT = NW * RPT
EPAD = RT * CH
NPAD = 10240
RSUB = NPAD // NS
DUMMY = N

_mesh = plsc.VectorSubcoreMesh(core_axis_name="c", subcore_axis_name="s")
_sc_params = pltpu.CompilerParams(needs_layout_passes=False)
_f32 = jnp.float32
_i32 = jnp.int32


def _attn_body(asrc, adst, srcp, dstp,
               exo, deno, dego,
               as_v, ad_v, sidx, didx, ex_v, ones_v, zb_v,
               den_sh, deg_sh):
    cid = lax.axis_index("c")
    sid = lax.axis_index("s")
    wid = sid * NC + cid

    def zb(i, c):
        zb_v[pl.ds(i * 16, 16)] = jnp.zeros((16,), _f32)
        return c
    lax.fori_loop(0, RSUB // 16, zb, 0)
    sl = pl.ds(sid * RSUB, RSUB)
    pltpu.sync_copy(zb_v, den_sh.at[sl])
    pltpu.sync_copy(zb_v, deg_sh.at[sl])
    for j in range(CH // 16):
        ones_v[pl.ds(j * 16, 16)] = jnp.ones((16,), _f32)

    pltpu.sync_copy(asrc, as_v)
    pltpu.sync_copy(adst, ad_v)
    pltpu.sync_copy(srcp.at[pl.ds(wid * RPT, RPT)], sidx)
    pltpu.sync_copy(dstp.at[pl.ds(wid * RPT, RPT)], didx)
    plsc.subcore_barrier()

    def chunk(r, c):
        def lane(i, c2):
            s16 = sidx[r, pl.ds(i * 16, 16)]
            d16 = didx[r, pl.ds(i * 16, 16)]
            s = plsc.load_gather(as_v, [s16]) + plsc.load_gather(ad_v, [d16])
            e = jnp.where(s > 0.0, s, 0.2 * s)
            ex_v[r, pl.ds(i * 16, 16)] = jnp.exp(e)
            return c2
        lax.fori_loop(0, CH // 16, lane, 0)
        pltpu.sync_copy(ex_v.at[r], den_sh.at[didx.at[r]], add=True)
        pltpu.sync_copy(ones_v, deg_sh.at[didx.at[r]], add=True)
        return c
    lax.fori_loop(0, RPT, chunk, 0)

    pltpu.sync_copy(ex_v, exo.at[pl.ds(wid * RPT, RPT)])
    plsc.subcore_barrier()
    pltpu.sync_copy(den_sh.at[sl], deno.at[cid, sl])
    pltpu.sync_copy(deg_sh.at[sl], dego.at[cid, sl])


_attn = pl.kernel(
    _attn_body,
    mesh=_mesh,
    compiler_params=_sc_params,
    out_type=[
        jax.ShapeDtypeStruct((RT, CH), _f32),
        jax.ShapeDtypeStruct((NC, NPAD), _f32),
        jax.ShapeDtypeStruct((NC, NPAD), _f32),
    ],
    scratch_types=[
        pltpu.VMEM((NPAD,), _f32),
        pltpu.VMEM((NPAD,), _f32),
        pltpu.VMEM((RPT, CH), _i32),
        pltpu.VMEM((RPT, CH), _i32),
        pltpu.VMEM((RPT, CH), _f32),
        pltpu.VMEM((CH,), _f32),
        pltpu.VMEM((RSUB,), _f32),
        pltpu.VMEM_SHARED((NPAD,), _f32),
        pltpu.VMEM_SHARED((NPAD,), _f32),
    ],
)


def _make_wspmm(F):
    JW = F // 16

    def body(h, exi, srcp, dstp, part, sidx, didx, ex_v, rows, acc_sh, sem):
        cid = lax.axis_index("c")
        sid = lax.axis_index("s")
        wid = sid * NC + cid

        def zrow(e, c):
            for j in range(JW):
                rows[e, pl.ds(j * 16, 16)] = jnp.zeros((16,), _f32)
            return c
        lax.fori_loop(0, CH, zrow, 0)

        def zcp(t, c):
            pltpu.sync_copy(rows, acc_sh.at[pl.ds(sid * RSUB + t * CH, CH)])
            return c
        lax.fori_loop(0, RSUB // CH, zcp, 0)

        pltpu.sync_copy(srcp.at[pl.ds(wid * RPT, RPT)], sidx)
        pltpu.sync_copy(dstp.at[pl.ds(wid * RPT, RPT)], didx)
        pltpu.sync_copy(exi.at[pl.ds(wid * RPT, RPT)], ex_v)
        plsc.subcore_barrier()

        def chunk(r, c):
            pltpu.async_copy(h.at[sidx.at[r]], rows, sem).wait()

            def edge(e, c2):
                w = plsc.load_gather(
                    ex_v, [jnp.full((16,), r, _i32), jnp.full((16,), e, _i32)])
                for j in range(JW):
                    rows[e, pl.ds(j * 16, 16)] = rows[e, pl.ds(j * 16, 16)] * w
                return c2
            lax.fori_loop(0, CH, edge, 0)
            pltpu.sync_copy(rows, acc_sh.at[didx.at[r]], add=True)
            return c
        lax.fori_loop(0, RPT, chunk, 0)

        plsc.subcore_barrier()
        sl = pl.ds(sid * RSUB, RSUB)
        pltpu.sync_copy(acc_sh.at[sl], part.at[cid, sl])

    return pl.kernel(
        body,
        mesh=_mesh,
        compiler_params=_sc_params,
        out_type=jax.ShapeDtypeStruct((NC, NPAD, F), _f32),
        scratch_types=[
            pltpu.VMEM((RPT, CH), _i32),
            pltpu.VMEM((RPT, CH), _i32),
            pltpu.VMEM((RPT, CH), _f32),
            pltpu.VMEM((CH, F), _f32),
            pltpu.VMEM_SHARED((NPAD, F), _f32),
            pltpu.SemaphoreType.DMA,
        ],
    )


def _make_uspmm(F):
    JW = F // 16

    def body(zs, srcp, dstp, part, sidx, didx, rows, acc_sh, sem):
        cid = lax.axis_index("c")
        sid = lax.axis_index("s")
        wid = sid * NC + cid

        def zrow(e, c):
            for j in range(JW):
                rows[e, pl.ds(j * 16, 16)] = jnp.zeros((16,), _f32)
            return c
        lax.fori_loop(0, CH, zrow, 0)

        def zcp(t, c):
            pltpu.sync_copy(rows, acc_sh.at[pl.ds(sid * RSUB + t * CH, CH)])
            return c
        lax.fori_loop(0, RSUB // CH, zcp, 0)

        pltpu.sync_copy(srcp.at[pl.ds(wid * RPT, RPT)], sidx)
        pltpu.sync_copy(dstp.at[pl.ds(wid * RPT, RPT)], didx)
        plsc.subcore_barrier()

        def chunk(r, c):
            pltpu.async_copy(zs.at[sidx.at[r]], rows, sem).wait()
            pltpu.sync_copy(rows, acc_sh.at[didx.at[r]], add=True)
            return c
        lax.fori_loop(0, RPT, chunk, 0)

        plsc.subcore_barrier()
        sl = pl.ds(sid * RSUB, RSUB)
        pltpu.sync_copy(acc_sh.at[sl], part.at[cid, sl])

    return pl.kernel(
        body,
        mesh=_mesh,
        compiler_params=_sc_params,
        out_type=jax.ShapeDtypeStruct((NC, NPAD, F), _f32),
        scratch_types=[
            pltpu.VMEM((RPT, CH), _i32),
            pltpu.VMEM((RPT, CH), _i32),
            pltpu.VMEM((CH, F), _f32),
            pltpu.VMEM_SHARED((NPAD, F), _f32),
            pltpu.SemaphoreType.DMA,
        ],
    )


_BN = 2048


def _make_encode(FI, H):
    def body(x_ref, w_ref, a1_ref, a2_ref, h_ref, s1_ref, s2_ref):
        h = jnp.dot(x_ref[...], w_ref[...], preferred_element_type=_f32)
        h_ref[...] = h
        s1_ref[...] = jnp.sum(h * a1_ref[...], axis=1, keepdims=True)
        s2_ref[...] = jnp.sum(h * a2_ref[...], axis=1, keepdims=True)

    return pl.pallas_call(
        body,
        grid=(NPAD // _BN,),
        in_specs=[
            pl.BlockSpec((_BN, FI), lambda i: (i, 0)),
            pl.BlockSpec((FI, H), lambda i: (0, 0)),
            pl.BlockSpec((1, H), lambda i: (0, 0)),
            pl.BlockSpec((1, H), lambda i: (0, 0)),
        ],
        out_specs=[
            pl.BlockSpec((_BN, H), lambda i: (i, 0)),
            pl.BlockSpec((_BN, 1), lambda i: (i, 0)),
            pl.BlockSpec((_BN, 1), lambda i: (i, 0)),
        ],
        out_shape=[
            jax.ShapeDtypeStruct((NPAD, H), _f32),
            jax.ShapeDtypeStruct((NPAD, 1), _f32),
            jax.ShapeDtypeStruct((NPAD, 1), _f32),
        ],
    )


def _make_h0fin(F):
    def body(p0, p1, den, dinv, h0_ref, zs_ref):
        h0 = (p0[...] + p1[...]) / (den[...] + 1e-16)
        h0_ref[...] = h0
        zs_ref[...] = h0 * dinv[...]

    return pl.pallas_call(
        body,
        grid=(NPAD // _BN,),
        in_specs=[
            pl.BlockSpec((_BN, F), lambda i: (i, 0)),
            pl.BlockSpec((_BN, F), lambda i: (i, 0)),
            pl.BlockSpec((_BN, 1), lambda i: (i, 0)),
            pl.BlockSpec((_BN, 1), lambda i: (i, 0)),
        ],
        out_specs=[
            pl.BlockSpec((_BN, F), lambda i: (i, 0)),
            pl.BlockSpec((_BN, F), lambda i: (i, 0)),
        ],
        out_shape=[
            jax.ShapeDtypeStruct((NPAD, F), _f32),
            jax.ShapeDtypeStruct((NPAD, F), _f32),
        ],
    )


def _make_fma2(F, relu):
    def body(p0, p1, rs, b, o_ref):
        v = rs[...] * (p0[...] + p1[...]) + ALPHA * b[...]
        if relu:
            v = jnp.maximum(v, 0.0)
        o_ref[...] = v

    return pl.pallas_call(
        body,
        grid=(NPAD // _BN,),
        in_specs=[
            pl.BlockSpec((_BN, F), lambda i: (i, 0)),
            pl.BlockSpec((_BN, F), lambda i: (i, 0)),
            pl.BlockSpec((_BN, 1), lambda i: (i, 0)),
            pl.BlockSpec((_BN, F), lambda i: (i, 0)),
        ],
        out_specs=pl.BlockSpec((_BN, F), lambda i: (i, 0)),
        out_shape=jax.ShapeDtypeStruct((NPAD, F), _f32),
    )


_enc = _make_encode(F_IN, HID)
_wsp = _make_wspmm(HID)
_usp = _make_uspmm(HID)
_h0f = _make_h0fin(HID)
_mid = _make_fma2(HID, False)
_fin1 = _make_fma2(HID, True)
_fin2 = _make_fma2(HID, False)


def _layer(xp, W, a_s, a_d, srcp, dstp, fin, dinv_in):
    H = W.shape[1]
    h, s1, s2 = _enc(xp, W, a_s.reshape(1, H), a_d.reshape(1, H))
    ex, denp, degp = _attn(s1.reshape(NPAD), s2.reshape(NPAD), srcp, dstp)
    den = (denp[0] + denp[1]).reshape(NPAD, 1)
    if dinv_in is None:
        dinv = lax.rsqrt(degp[0] + degp[1] + 1.0)
    else:
        dinv = dinv_in
    dcol = dinv.reshape(NPAD, 1)
    p = _wsp(h, ex, srcp, dstp)
    h0, zs0 = _h0f(p[0], p[1], den, dcol)
    c1 = ((1.0 - ALPHA) * dinv * dinv).reshape(NPAD, 1)
    rfin = ((1.0 - ALPHA) * dinv).reshape(NPAD, 1)
    zs = zs0
    for t in range(K):
        q = _usp(zs, srcp, dstp)
        if t < K - 1:
            zs = _mid(q[0], q[1], c1, zs0)
        else:
            z = fin(q[0], q[1], rfin, h0)
    return z, dinv


def kernel(x, edge_index, W1, a_src1, a_dst1, W2, a_src2, a_dst2):
    src = edge_index[0]
    dst = edge_index[1]
    pad = jnp.full((EPAD - E,), DUMMY, _i32)
    srcp = jnp.concatenate([src, pad]).reshape(RT, CH)
    dstp = jnp.concatenate([dst, pad]).reshape(RT, CH)
    xp = jnp.zeros((NPAD, F_IN), _f32).at[:N].set(x)
    W2p = jnp.zeros((HID, HID), _f32).at[:, :OUT].set(W2)
    a2sp = jnp.zeros((HID,), _f32).at[:OUT].set(a_src2)
    a2dp = jnp.zeros((HID,), _f32).at[:OUT].set(a_dst2)
    z1, dinv = _layer(xp, W1, a_src1, a_dst1, srcp, dstp, _fin1, None)
    z2, _ = _layer(z1, W2p, a2sp, a2dp, srcp, dstp, _fin2, dinv)
    return z2[:N, :OUT]


# speedup vs baseline: 5.3563x; 1.1006x over previous
import functools

import jax
import jax.numpy as jnp
from jax import lax
from jax.experimental import pallas as pl
from jax.experimental.pallas import tpu as pltpu
from jax.experimental.pallas import tpu_sc as plsc

N = 10000
E = 320000
F_IN = 128
HID = 128
OUT = 64
ALPHA = 0.1
K = 10

NC = 2
NS = 16
NW = NC * NS
CH = 128
RPT = (-(-E // (NW * CH)) + 7) // 8 * 8
RT = NW * RPT
EPAD = RT * CH
NPAD = 10240
RSUB = NPAD // NS
DUMMY = N

_mesh = plsc.VectorSubcoreMesh(core_axis_name="c", subcore_axis_name="s")
_sc_params = pltpu.CompilerParams(needs_layout_passes=False)
_f32 = jnp.float32
_i32 = jnp.int32


def _attn_body(asrc, adst, srcp, dstp,
               exo, deno, dego,
               as_v, ad_v, sidx, didx, ex_v, ones_v, zb_v,
               den_sh, deg_sh):
    cid = lax.axis_index("c")
    sid = lax.axis_index("s")
    wid = sid * NC + cid

    def zb(i, c):
        zb_v[pl.ds(i * 16, 16)] = jnp.zeros((16,), _f32)
        return c
    lax.fori_loop(0, RSUB // 16, zb, 0)
    sl = pl.ds(sid * RSUB, RSUB)
    pltpu.sync_copy(zb_v, den_sh.at[sl])
    pltpu.sync_copy(zb_v, deg_sh.at[sl])
    for j in range(CH // 16):
        ones_v[pl.ds(j * 16, 16)] = jnp.ones((16,), _f32)

    pltpu.sync_copy(asrc, as_v)
    pltpu.sync_copy(adst, ad_v)
    pltpu.sync_copy(srcp.at[pl.ds(wid * RPT, RPT)], sidx)
    pltpu.sync_copy(dstp.at[pl.ds(wid * RPT, RPT)], didx)
    plsc.subcore_barrier()

    def chunk(r, c):
        def lane(i, c2):
            s16 = sidx[r, pl.ds(i * 16, 16)]
            d16 = didx[r, pl.ds(i * 16, 16)]
            s = plsc.load_gather(as_v, [s16]) + plsc.load_gather(ad_v, [d16])
            e = jnp.where(s > 0.0, s, 0.2 * s)
            ex_v[r, pl.ds(i * 16, 16)] = jnp.exp(e)
            return c2
        lax.fori_loop(0, CH // 16, lane, 0)
        pltpu.sync_copy(ex_v.at[r], den_sh.at[didx.at[r]], add=True)
        pltpu.sync_copy(ones_v, deg_sh.at[didx.at[r]], add=True)
        return c
    lax.fori_loop(0, RPT, chunk, 0)

    pltpu.sync_copy(ex_v, exo.at[pl.ds(wid * RPT, RPT)])
    plsc.subcore_barrier()
    pltpu.sync_copy(den_sh.at[sl], deno.at[cid, sl])
    pltpu.sync_copy(deg_sh.at[sl], dego.at[cid, sl])


_attn = pl.kernel(
    _attn_body,
    mesh=_mesh,
    compiler_params=_sc_params,
    out_type=[
        jax.ShapeDtypeStruct((RT, CH), _f32),
        jax.ShapeDtypeStruct((NC, NPAD), _f32),
        jax.ShapeDtypeStruct((NC, NPAD), _f32),
    ],
    scratch_types=[
        pltpu.VMEM((NPAD,), _f32),
        pltpu.VMEM((NPAD,), _f32),
        pltpu.VMEM((RPT, CH), _i32),
        pltpu.VMEM((RPT, CH), _i32),
        pltpu.VMEM((RPT, CH), _f32),
        pltpu.VMEM((CH,), _f32),
        pltpu.VMEM((RSUB,), _f32),
        pltpu.VMEM_SHARED((NPAD,), _f32),
        pltpu.VMEM_SHARED((NPAD,), _f32),
    ],
)


def _make_wspmm(F):
    JW = F // 16

    def body(h, exi, srcp, dstp, part, sidx, didx, ex_v, rows, acc_sh, sem):
        cid = lax.axis_index("c")
        sid = lax.axis_index("s")
        wid = sid * NC + cid

        def zrow(e, c):
            for j in range(JW):
                rows[e, pl.ds(j * 16, 16)] = jnp.zeros((16,), _f32)
            return c
        lax.fori_loop(0, CH, zrow, 0)

        def zcp(t, c):
            pltpu.sync_copy(rows, acc_sh.at[pl.ds(sid * RSUB + t * CH, CH)])
            return c
        lax.fori_loop(0, RSUB // CH, zcp, 0)

        pltpu.sync_copy(srcp.at[pl.ds(wid * RPT, RPT)], sidx)
        pltpu.sync_copy(dstp.at[pl.ds(wid * RPT, RPT)], didx)
        pltpu.sync_copy(exi.at[pl.ds(wid * RPT, RPT)], ex_v)
        plsc.subcore_barrier()

        def chunk(r, c):
            pltpu.async_copy(h.at[sidx.at[r]], rows, sem).wait()

            def edge(e, c2):
                w = plsc.load_gather(
                    ex_v, [jnp.full((16,), r, _i32), jnp.full((16,), e, _i32)])
                for j in range(JW):
                    rows[e, pl.ds(j * 16, 16)] = rows[e, pl.ds(j * 16, 16)] * w
                return c2
            lax.fori_loop(0, CH, edge, 0)
            pltpu.sync_copy(rows, acc_sh.at[didx.at[r]], add=True)
            return c
        lax.fori_loop(0, RPT, chunk, 0)

        plsc.subcore_barrier()
        sl = pl.ds(sid * RSUB, RSUB)
        pltpu.sync_copy(acc_sh.at[sl], part.at[cid, sl])

    return pl.kernel(
        body,
        mesh=_mesh,
        compiler_params=_sc_params,
        out_type=jax.ShapeDtypeStruct((NC, NPAD, F), _f32),
        scratch_types=[
            pltpu.VMEM((RPT, CH), _i32),
            pltpu.VMEM((RPT, CH), _i32),
            pltpu.VMEM((RPT, CH), _f32),
            pltpu.VMEM((CH, F), _f32),
            pltpu.VMEM_SHARED((NPAD, F), _f32),
            pltpu.SemaphoreType.DMA,
        ],
    )


def _make_uspmm(F):
    GR = 8
    NG = RPT // GR

    def body(zs, ep, zer, part, idxb, rows, acc_sh, *sems):
        sg = sems[0:2]
        ss = sems[2:4]
        si = sems[4:6]
        cid = lax.axis_index("c")
        sid = lax.axis_index("s")
        wid = sid * NC + cid
        base = wid * RPT

        pltpu.sync_copy(zer, acc_sh.at[pl.ds(sid * RSUB, RSUB)])

        def idx_start(g):
            pltpu.async_copy(
                ep.at[pl.ds(base + g * GR, GR)], idxb.at[g % 2], si[g % 2])

        def idx_wait(g):
            pltpu.make_async_copy(
                ep.at[pl.ds(base + g * GR, GR)], idxb.at[g % 2],
                si[g % 2]).wait()

        def src_ref(c):
            return idxb.at[(c // GR) % 2, c % GR, 0]

        def dst_ref(c):
            return idxb.at[(c // GR) % 2, c % GR, 1]

        def gather_start(c):
            pltpu.async_copy(zs.at[src_ref(c)], rows.at[c % 2], sg[c % 2])

        def gather_wait(c):
            pltpu.make_async_copy(
                zs.at[src_ref(c)], rows.at[c % 2], sg[c % 2]).wait()

        def scat_start(c):
            pltpu.async_copy(
                rows.at[c % 2], acc_sh.at[dst_ref(c)], ss[c % 2], add=True)

        def scat_wait(c):
            pltpu.make_async_copy(
                rows.at[c % 2], acc_sh.at[dst_ref(c)], ss[c % 2]).wait()

        idx_start(0)
        plsc.subcore_barrier()
        for g in range(NG):
            idx_wait(g)
            for j in range(GR):
                c = g * GR + j
                if c >= 2:
                    scat_wait(c - 2)
                if j == 2 and g + 1 < NG:
                    idx_start(g + 1)
                gather_start(c)
                if c >= 1:
                    gather_wait(c - 1)
                    scat_start(c - 1)
        gather_wait(RPT - 1)
        scat_start(RPT - 1)
        scat_wait(RPT - 2)
        scat_wait(RPT - 1)
        plsc.subcore_barrier()
        sl = pl.ds(sid * RSUB, RSUB)
        pltpu.sync_copy(acc_sh.at[sl], part.at[cid, sl])

    return pl.kernel(
        body,
        mesh=_mesh,
        compiler_params=_sc_params,
        out_type=jax.ShapeDtypeStruct((NC, NPAD, F), _f32),
        scratch_types=[
            pltpu.VMEM((2, GR, 2, CH), _i32),
            pltpu.VMEM((2, CH, F), _f32),
            pltpu.VMEM_SHARED((NPAD, F), _f32),
        ] + [pltpu.SemaphoreType.DMA] * 6,
    )


_BN = 2048


def _make_encode(FI, H):
    def body(x_ref, w_ref, a1_ref, a2_ref, h_ref, s1_ref, s2_ref):
        h = jnp.dot(x_ref[...], w_ref[...], preferred_element_type=_f32)
        h_ref[...] = h
        s1_ref[...] = jnp.sum(h * a1_ref[...], axis=1, keepdims=True)
        s2_ref[...] = jnp.sum(h * a2_ref[...], axis=1, keepdims=True)

    return pl.pallas_call(
        body,
        grid=(NPAD // _BN,),
        in_specs=[
            pl.BlockSpec((_BN, FI), lambda i: (i, 0)),
            pl.BlockSpec((FI, H), lambda i: (0, 0)),
            pl.BlockSpec((1, H), lambda i: (0, 0)),
            pl.BlockSpec((1, H), lambda i: (0, 0)),
        ],
        out_specs=[
            pl.BlockSpec((_BN, H), lambda i: (i, 0)),
            pl.BlockSpec((_BN, 1), lambda i: (i, 0)),
            pl.BlockSpec((_BN, 1), lambda i: (i, 0)),
        ],
        out_shape=[
            jax.ShapeDtypeStruct((NPAD, H), _f32),
            jax.ShapeDtypeStruct((NPAD, 1), _f32),
            jax.ShapeDtypeStruct((NPAD, 1), _f32),
        ],
    )


def _make_h0fin(F):
    def body(p0, p1, den, dinv, h0_ref, zs_ref):
        h0 = (p0[...] + p1[...]) / (den[...] + 1e-16)
        h0_ref[...] = h0
        zs_ref[...] = h0 * dinv[...]

    return pl.pallas_call(
        body,
        grid=(NPAD // _BN,),
        in_specs=[
            pl.BlockSpec((_BN, F), lambda i: (i, 0)),
            pl.BlockSpec((_BN, F), lambda i: (i, 0)),
            pl.BlockSpec((_BN, 1), lambda i: (i, 0)),
            pl.BlockSpec((_BN, 1), lambda i: (i, 0)),
        ],
        out_specs=[
            pl.BlockSpec((_BN, F), lambda i: (i, 0)),
            pl.BlockSpec((_BN, F), lambda i: (i, 0)),
        ],
        out_shape=[
            jax.ShapeDtypeStruct((NPAD, F), _f32),
            jax.ShapeDtypeStruct((NPAD, F), _f32),
        ],
    )


def _make_fma2(F, relu):
    def body(p0, p1, rs, b, o_ref):
        v = rs[...] * (p0[...] + p1[...]) + ALPHA * b[...]
        if relu:
            v = jnp.maximum(v, 0.0)
        o_ref[...] = v

    return pl.pallas_call(
        body,
        grid=(NPAD // _BN,),
        in_specs=[
            pl.BlockSpec((_BN, F), lambda i: (i, 0)),
            pl.BlockSpec((_BN, F), lambda i: (i, 0)),
            pl.BlockSpec((_BN, 1), lambda i: (i, 0)),
            pl.BlockSpec((_BN, F), lambda i: (i, 0)),
        ],
        out_specs=pl.BlockSpec((_BN, F), lambda i: (i, 0)),
        out_shape=jax.ShapeDtypeStruct((NPAD, F), _f32),
    )


_enc = _make_encode(F_IN, HID)
_wsp = _make_wspmm(HID)
_usp = _make_uspmm(HID)
_h0f = _make_h0fin(HID)
_mid = _make_fma2(HID, False)
_fin1 = _make_fma2(HID, True)
_fin2 = _make_fma2(HID, False)


def _layer(xp, W, a_s, a_d, srcp, dstp, ep, zer, fin, dinv_in):
    H = W.shape[1]
    h, s1, s2 = _enc(xp, W, a_s.reshape(1, H), a_d.reshape(1, H))
    ex, denp, degp = _attn(s1.reshape(NPAD), s2.reshape(NPAD), srcp, dstp)
    den = (denp[0] + denp[1]).reshape(NPAD, 1)
    if dinv_in is None:
        dinv = lax.rsqrt(degp[0] + degp[1] + 1.0)
    else:
        dinv = dinv_in
    dcol = dinv.reshape(NPAD, 1)
    p = _wsp(h, ex, srcp, dstp)
    h0, zs0 = _h0f(p[0], p[1], den, dcol)
    c1 = ((1.0 - ALPHA) * dinv * dinv).reshape(NPAD, 1)
    rfin = ((1.0 - ALPHA) * dinv).reshape(NPAD, 1)
    zs = zs0
    for t in range(K):
        q = _usp(zs, ep, zer)
        if t < K - 1:
            zs = _mid(q[0], q[1], c1, zs0)
        else:
            z = fin(q[0], q[1], rfin, h0)
    return z, dinv


def kernel(x, edge_index, W1, a_src1, a_dst1, W2, a_src2, a_dst2):
    src = edge_index[0]
    dst = edge_index[1]
    pad = jnp.full((EPAD - E,), DUMMY, _i32)
    srcp = jnp.concatenate([src, pad]).reshape(RT, CH)
    dstp = jnp.concatenate([dst, pad]).reshape(RT, CH)
    xp = jnp.zeros((NPAD, F_IN), _f32).at[:N].set(x)
    W2p = jnp.zeros((HID, HID), _f32).at[:, :OUT].set(W2)
    a2sp = jnp.zeros((HID,), _f32).at[:OUT].set(a_src2)
    a2dp = jnp.zeros((HID,), _f32).at[:OUT].set(a_dst2)
    ep = jnp.stack([srcp, dstp], axis=1)
    zer = jnp.zeros((RSUB, HID), _f32)
    z1, dinv = _layer(xp, W1, a_src1, a_dst1, srcp, dstp, ep, zer, _fin1, None)
    z2, _ = _layer(z1, W2p, a2sp, a2dp, srcp, dstp, ep, zer, _fin2, dinv)
    return z2[:N, :OUT]


# speedup vs baseline: 5.8232x; 1.0872x over previous
import functools

import jax
import jax.numpy as jnp
from jax import lax
from jax.experimental import pallas as pl
from jax.experimental.pallas import tpu as pltpu
from jax.experimental.pallas import tpu_sc as plsc

N = 10000
E = 320000
F_IN = 128
HID = 128
OUT = 64
ALPHA = 0.1
K = 10

NC = 2
NS = 16
NW = NC * NS
CH = 128
RPT = (-(-E // (NW * CH)) + 7) // 8 * 8
RT = NW * RPT
EPAD = RT * CH
NPAD = 10240
RSUB = NPAD // NS
DUMMY = N

_mesh = plsc.VectorSubcoreMesh(core_axis_name="c", subcore_axis_name="s")
_sc_params = pltpu.CompilerParams(needs_layout_passes=False)
_sc_params_sp = pltpu.CompilerParams(
    needs_layout_passes=False, use_tc_tiling_on_sc=False)
_f32 = jnp.float32
_i32 = jnp.int32


def _attn_body(asrc, adst, srcp, dstp,
               exo, deno, dego,
               as_v, ad_v, sidx, didx, ex_v, ones_v, zb_v,
               den_sh, deg_sh):
    cid = lax.axis_index("c")
    sid = lax.axis_index("s")
    wid = sid * NC + cid

    def zb(i, c):
        zb_v[pl.ds(i * 16, 16)] = jnp.zeros((16,), _f32)
        return c
    lax.fori_loop(0, RSUB // 16, zb, 0)
    sl = pl.ds(sid * RSUB, RSUB)
    pltpu.sync_copy(zb_v, den_sh.at[sl])
    pltpu.sync_copy(zb_v, deg_sh.at[sl])
    for j in range(CH // 16):
        ones_v[pl.ds(j * 16, 16)] = jnp.ones((16,), _f32)

    pltpu.sync_copy(asrc, as_v)
    pltpu.sync_copy(adst, ad_v)
    pltpu.sync_copy(srcp.at[pl.ds(wid * RPT, RPT)], sidx)
    pltpu.sync_copy(dstp.at[pl.ds(wid * RPT, RPT)], didx)
    plsc.subcore_barrier()

    def chunk(r, c):
        def lane(i, c2):
            s16 = sidx[r, pl.ds(i * 16, 16)]
            d16 = didx[r, pl.ds(i * 16, 16)]
            s = plsc.load_gather(as_v, [s16]) + plsc.load_gather(ad_v, [d16])
            e = jnp.where(s > 0.0, s, 0.2 * s)
            ex_v[r, pl.ds(i * 16, 16)] = jnp.exp(e)
            return c2
        lax.fori_loop(0, CH // 16, lane, 0)
        pltpu.sync_copy(ex_v.at[r], den_sh.at[didx.at[r]], add=True)
        pltpu.sync_copy(ones_v, deg_sh.at[didx.at[r]], add=True)
        return c
    lax.fori_loop(0, RPT, chunk, 0)

    pltpu.sync_copy(ex_v, exo.at[pl.ds(wid * RPT, RPT)])
    plsc.subcore_barrier()
    pltpu.sync_copy(den_sh.at[sl], deno.at[cid, sl])
    pltpu.sync_copy(deg_sh.at[sl], dego.at[cid, sl])


_attn = pl.kernel(
    _attn_body,
    mesh=_mesh,
    compiler_params=_sc_params,
    out_type=[
        jax.ShapeDtypeStruct((RT, CH), _f32),
        jax.ShapeDtypeStruct((NC, NPAD), _f32),
        jax.ShapeDtypeStruct((NC, NPAD), _f32),
    ],
    scratch_types=[
        pltpu.VMEM((NPAD,), _f32),
        pltpu.VMEM((NPAD,), _f32),
        pltpu.VMEM((RPT, CH), _i32),
        pltpu.VMEM((RPT, CH), _i32),
        pltpu.VMEM((RPT, CH), _f32),
        pltpu.VMEM((CH,), _f32),
        pltpu.VMEM((RSUB,), _f32),
        pltpu.VMEM_SHARED((NPAD,), _f32),
        pltpu.VMEM_SHARED((NPAD,), _f32),
    ],
)


def _make_spmm(F, weighted, NB, GR, params):
    NG = RPT // GR
    JW = F // 16

    def body(zs, *refs):
        if weighted:
            (exi, ep, zer, part, ex_v, idxb, rows, acc_sh, *sems) = refs
        else:
            (ep, zer, part, idxb, rows, acc_sh, *sems) = refs
        sg = sems[0:NB]
        ss = sems[NB:2 * NB]
        si = sems[2 * NB:]
        cid = lax.axis_index("c")
        sid = lax.axis_index("s")
        wid = sid * NC + cid
        base = wid * RPT

        pltpu.sync_copy(zer, acc_sh.at[pl.ds(sid * RSUB, RSUB)])
        if weighted:
            pltpu.sync_copy(exi.at[pl.ds(base, RPT)], ex_v)

        def idx_start(g):
            pltpu.async_copy(
                ep.at[pl.ds(base + g * GR, GR)], idxb.at[g % 2], si[g % 2])

        def idx_wait(g):
            pltpu.make_async_copy(
                ep.at[pl.ds(base + g * GR, GR)], idxb.at[g % 2],
                si[g % 2]).wait()

        def src_ref(c):
            return idxb.at[(c // GR) % 2, c % GR, 0]

        def dst_ref(c):
            return idxb.at[(c // GR) % 2, c % GR, 1]

        def gather_start(c):
            pltpu.async_copy(zs.at[src_ref(c)], rows.at[c % NB], sg[c % NB])

        def gather_wait(c):
            pltpu.make_async_copy(
                zs.at[src_ref(c)], rows.at[c % NB], sg[c % NB]).wait()

        def scale(c):
            b = c % NB

            def edge(e, carry):
                w = plsc.load_gather(
                    ex_v, [jnp.full((16,), c, _i32), jnp.full((16,), e, _i32)])
                for j in range(JW):
                    rows[b, e, pl.ds(j * 16, 16)] = (
                        rows[b, e, pl.ds(j * 16, 16)] * w)
                return carry
            lax.fori_loop(0, CH, edge, 0)

        def scat_start(c):
            pltpu.async_copy(
                rows.at[c % NB], acc_sh.at[dst_ref(c)], ss[c % NB], add=True)

        def scat_wait(c):
            pltpu.make_async_copy(
                rows.at[c % NB], acc_sh.at[dst_ref(c)], ss[c % NB]).wait()

        def stage2(c):
            gather_wait(c)
            if weighted:
                scale(c)
            scat_start(c)

        idx_start(0)
        plsc.subcore_barrier()
        for g in range(NG):
            idx_wait(g)
            for j in range(GR):
                c = g * GR + j
                if c >= NB:
                    scat_wait(c - NB)
                if j == min(NB, GR - 1) and g + 1 < NG:
                    idx_start(g + 1)
                gather_start(c)
                if c >= 1:
                    stage2(c - 1)
        stage2(RPT - 1)
        for c in range(RPT - NB, RPT):
            scat_wait(c)
        plsc.subcore_barrier()
        sl = pl.ds(sid * RSUB, RSUB)
        pltpu.sync_copy(acc_sh.at[sl], part.at[cid, sl])

    scratch = []
    if weighted:
        scratch.append(pltpu.VMEM((RPT, CH), _f32))
    scratch += [
        pltpu.VMEM((2, GR, 2, CH), _i32),
        pltpu.VMEM((NB, CH, F), _f32),
        pltpu.VMEM_SHARED((NPAD, F), _f32),
    ] + [pltpu.SemaphoreType.DMA] * (2 * NB + 2)

    return pl.kernel(
        body,
        mesh=_mesh,
        compiler_params=params,
        out_type=jax.ShapeDtypeStruct((NC, NPAD, F), _f32),
        scratch_types=scratch,
    )


_BN = 2048


def _make_encode(FI, H):
    def body(x_ref, w_ref, a1_ref, a2_ref, h_ref, s1_ref, s2_ref):
        h = jnp.dot(x_ref[...], w_ref[...], preferred_element_type=_f32)
        h_ref[...] = h
        s1_ref[...] = jnp.sum(h * a1_ref[...], axis=1, keepdims=True)
        s2_ref[...] = jnp.sum(h * a2_ref[...], axis=1, keepdims=True)

    return pl.pallas_call(
        body,
        grid=(NPAD // _BN,),
        in_specs=[
            pl.BlockSpec((_BN, FI), lambda i: (i, 0)),
            pl.BlockSpec((FI, H), lambda i: (0, 0)),
            pl.BlockSpec((1, H), lambda i: (0, 0)),
            pl.BlockSpec((1, H), lambda i: (0, 0)),
        ],
        out_specs=[
            pl.BlockSpec((_BN, H), lambda i: (i, 0)),
            pl.BlockSpec((_BN, 1), lambda i: (i, 0)),
            pl.BlockSpec((_BN, 1), lambda i: (i, 0)),
        ],
        out_shape=[
            jax.ShapeDtypeStruct((NPAD, H), _f32),
            jax.ShapeDtypeStruct((NPAD, 1), _f32),
            jax.ShapeDtypeStruct((NPAD, 1), _f32),
        ],
    )


def _make_h0fin(F):
    def body(p0, p1, den, dinv, h0_ref, zs_ref):
        h0 = (p0[...] + p1[...]) / (den[...] + 1e-16)
        h0_ref[...] = h0
        zs_ref[...] = h0 * dinv[...]

    return pl.pallas_call(
        body,
        grid=(NPAD // _BN,),
        in_specs=[
            pl.BlockSpec((_BN, F), lambda i: (i, 0)),
            pl.BlockSpec((_BN, F), lambda i: (i, 0)),
            pl.BlockSpec((_BN, 1), lambda i: (i, 0)),
            pl.BlockSpec((_BN, 1), lambda i: (i, 0)),
        ],
        out_specs=[
            pl.BlockSpec((_BN, F), lambda i: (i, 0)),
            pl.BlockSpec((_BN, F), lambda i: (i, 0)),
        ],
        out_shape=[
            jax.ShapeDtypeStruct((NPAD, F), _f32),
            jax.ShapeDtypeStruct((NPAD, F), _f32),
        ],
    )


def _make_fma2(F, relu):
    def body(p0, p1, rs, b, o_ref):
        v = rs[...] * (p0[...] + p1[...]) + ALPHA * b[...]
        if relu:
            v = jnp.maximum(v, 0.0)
        o_ref[...] = v

    return pl.pallas_call(
        body,
        grid=(NPAD // _BN,),
        in_specs=[
            pl.BlockSpec((_BN, F), lambda i: (i, 0)),
            pl.BlockSpec((_BN, F), lambda i: (i, 0)),
            pl.BlockSpec((_BN, 1), lambda i: (i, 0)),
            pl.BlockSpec((_BN, F), lambda i: (i, 0)),
        ],
        out_specs=pl.BlockSpec((_BN, F), lambda i: (i, 0)),
        out_shape=jax.ShapeDtypeStruct((NPAD, F), _f32),
    )


_enc1 = _make_encode(F_IN, HID)
_enc2 = _make_encode(HID, OUT)
_usp = _make_spmm(OUT, False, 8, 16, _sc_params_sp)
_wsp = _make_spmm(OUT, True, 4, 16, _sc_params_sp)
_h0f = _make_h0fin(OUT)
_mid = _make_fma2(OUT, False)
_fin1 = _make_fma2(OUT, True)
_fin2 = _make_fma2(OUT, False)


def _layer(halves, s1, s2, srcp, dstp, ep, zer, fin, dinv_in):
    ex, denp, degp = _attn(s1.reshape(NPAD), s2.reshape(NPAD), srcp, dstp)
    den = (denp[0] + denp[1]).reshape(NPAD, 1)
    if dinv_in is None:
        dinv = lax.rsqrt(degp[0] + degp[1] + 1.0)
    else:
        dinv = dinv_in
    dcol = dinv.reshape(NPAD, 1)
    c1 = ((1.0 - ALPHA) * dinv * dinv).reshape(NPAD, 1)
    rfin = ((1.0 - ALPHA) * dinv).reshape(NPAD, 1)
    h0s, zss, zs0s = [], [], []
    for hh in halves:
        p = _wsp(hh, ex, ep, zer)
        h0_i, zs_i = _h0f(p[0], p[1], den, dcol)
        h0s.append(h0_i)
        zs0s.append(zs_i)
        zss.append(zs_i)
    outs = [None] * len(halves)
    for t in range(K):
        for i in range(len(halves)):
            q = _usp(zss[i], ep, zer)
            if t < K - 1:
                zss[i] = _mid(q[0], q[1], c1, zs0s[i])
            else:
                outs[i] = fin(q[0], q[1], rfin, h0s[i])
    return outs, dinv


def kernel(x, edge_index, W1, a_src1, a_dst1, W2, a_src2, a_dst2):
    src = edge_index[0]
    dst = edge_index[1]
    pad = jnp.full((EPAD - E,), DUMMY, _i32)
    srcp = jnp.concatenate([src, pad]).reshape(RT, CH)
    dstp = jnp.concatenate([dst, pad]).reshape(RT, CH)
    ep = jnp.stack([srcp, dstp], axis=1)
    zer = jnp.zeros((RSUB, OUT), _f32)
    xp = jnp.zeros((NPAD, F_IN), _f32).at[:N].set(x)
    h1, s11, s12 = _enc1(xp, W1, a_src1.reshape(1, HID), a_dst1.reshape(1, HID))
    halves1 = [h1[:, :OUT], h1[:, OUT:]]
    outs1, dinv = _layer(halves1, s11, s12, srcp, dstp, ep, zer, _fin1, None)
    x2 = jnp.concatenate(outs1, axis=1)
    h2, s21, s22 = _enc2(x2, W2, a_src2.reshape(1, OUT), a_dst2.reshape(1, OUT))
    outs2, _ = _layer([h2], s21, s22, srcp, dstp, ep, zer, _fin2, dinv)
    return outs2[0][:N]


# speedup vs baseline: 6.2697x; 1.0767x over previous
import functools

import jax
import jax.numpy as jnp
from jax import lax
from jax.experimental import pallas as pl
from jax.experimental.pallas import tpu as pltpu
from jax.experimental.pallas import tpu_sc as plsc

N = 10000
E = 320000
F_IN = 128
HID = 128
OUT = 64
ALPHA = 0.1
K = 10

NC = 2
NS = 16
NW = NC * NS
CH = 128
RPT = (-(-E // (NW * CH)) + 7) // 8 * 8
RT = NW * RPT
EPAD = RT * CH
NPAD = 10240
RSUB = NPAD // NS
DUMMY = N

_mesh = plsc.VectorSubcoreMesh(core_axis_name="c", subcore_axis_name="s")
_sc_params = pltpu.CompilerParams(needs_layout_passes=False)
_sc_params_sp = pltpu.CompilerParams(
    needs_layout_passes=False, use_tc_tiling_on_sc=False)
_f32 = jnp.float32
_i32 = jnp.int32


def _attn_body(asrc, adst, srcp, dstp,
               exo, deno, dego,
               as_v, ad_v, sidx, didx, ex_v, ones_v, zb_v,
               den_sh, deg_sh, *sems):
    NS4 = 4
    se = sems[0:NS4]
    so = sems[NS4:]
    cid = lax.axis_index("c")
    sid = lax.axis_index("s")
    wid = sid * NC + cid

    def zb(i, c):
        zb_v[pl.ds(i * 16, 16)] = jnp.zeros((16,), _f32)
        return c
    lax.fori_loop(0, RSUB // 16, zb, 0)
    sl = pl.ds(sid * RSUB, RSUB)
    pltpu.sync_copy(zb_v, den_sh.at[sl])
    pltpu.sync_copy(zb_v, deg_sh.at[sl])
    for j in range(CH // 16):
        ones_v[pl.ds(j * 16, 16)] = jnp.ones((16,), _f32)

    pltpu.sync_copy(asrc, as_v)
    pltpu.sync_copy(adst, ad_v)
    pltpu.sync_copy(srcp.at[pl.ds(wid * RPT, RPT)], sidx)
    pltpu.sync_copy(dstp.at[pl.ds(wid * RPT, RPT)], didx)
    plsc.subcore_barrier()

    def scat_ex_start(r, b):
        pltpu.async_copy(ex_v.at[r], den_sh.at[didx.at[r]], se[b], add=True)

    def scat_ex_wait(r, b):
        pltpu.make_async_copy(
            ex_v.at[r], den_sh.at[didx.at[r]], se[b]).wait()

    def scat_on_start(r, b):
        pltpu.async_copy(ones_v, deg_sh.at[didx.at[r]], so[b], add=True)

    def scat_on_wait(r, b):
        pltpu.make_async_copy(
            ones_v, deg_sh.at[didx.at[r]], so[b]).wait()

    def compute(r):
        def lane(i, c2):
            s16 = sidx[r, pl.ds(i * 16, 16)]
            d16 = didx[r, pl.ds(i * 16, 16)]
            sv = plsc.load_gather(as_v, [s16]) + plsc.load_gather(ad_v, [d16])
            e = jnp.where(sv > 0.0, sv, 0.2 * sv)
            ex_v[r, pl.ds(i * 16, 16)] = jnp.exp(e)
            return c2
        lax.fori_loop(0, CH // 16, lane, 0)

    for b in range(NS4):
        compute(b)
        scat_ex_start(b, b)
        scat_on_start(b, b)

    def group(g, c):
        for b in range(NS4):
            r = g * NS4 + b
            compute(r)
            scat_ex_wait(r - NS4, b)
            scat_on_wait(r - NS4, b)
            scat_ex_start(r, b)
            scat_on_start(r, b)
        return c
    lax.fori_loop(1, RPT // NS4, group, 0)
    for b in range(NS4):
        r = RPT - NS4 + b
        scat_ex_wait(r, b)
        scat_on_wait(r, b)

    pltpu.sync_copy(ex_v, exo.at[pl.ds(wid * RPT, RPT)])
    plsc.subcore_barrier()
    pltpu.sync_copy(den_sh.at[sl], deno.at[cid, sl])
    pltpu.sync_copy(deg_sh.at[sl], dego.at[cid, sl])


_attn = pl.kernel(
    _attn_body,
    mesh=_mesh,
    compiler_params=_sc_params,
    out_type=[
        jax.ShapeDtypeStruct((RT, CH), _f32),
        jax.ShapeDtypeStruct((NC, NPAD), _f32),
        jax.ShapeDtypeStruct((NC, NPAD), _f32),
    ],
    scratch_types=[
        pltpu.VMEM((NPAD,), _f32),
        pltpu.VMEM((NPAD,), _f32),
        pltpu.VMEM((RPT, CH), _i32),
        pltpu.VMEM((RPT, CH), _i32),
        pltpu.VMEM((RPT, CH), _f32),
        pltpu.VMEM((CH,), _f32),
        pltpu.VMEM((RSUB,), _f32),
        pltpu.VMEM_SHARED((NPAD,), _f32),
        pltpu.VMEM_SHARED((NPAD,), _f32),
    ] + [pltpu.SemaphoreType.DMA] * 8,
)


def _make_spmm(F, weighted, NB, GR, params):
    NG = RPT // GR
    JW = F // 16

    def body(zs, *refs):
        if weighted:
            (exi, ep, zer, part, ex_v, idxb, rows, acc_sh, *sems) = refs
        else:
            (ep, zer, part, idxb, rows, acc_sh, *sems) = refs
        sg = sems[0:NB]
        ss = sems[NB:2 * NB]
        si = sems[2 * NB:]
        cid = lax.axis_index("c")
        sid = lax.axis_index("s")
        wid = sid * NC + cid
        base = wid * RPT

        pltpu.sync_copy(zer, acc_sh.at[pl.ds(sid * RSUB, RSUB)])
        if weighted:
            pltpu.sync_copy(exi.at[pl.ds(base, RPT)], ex_v)

        def idx_start(g):
            pltpu.async_copy(
                ep.at[pl.ds(base + g * GR, GR)], idxb.at[g % 2], si[g % 2])

        def idx_wait(g):
            pltpu.make_async_copy(
                ep.at[pl.ds(base + g * GR, GR)], idxb.at[g % 2],
                si[g % 2]).wait()

        def src_ref(c):
            return idxb.at[(c // GR) % 2, c % GR, 0]

        def dst_ref(c):
            return idxb.at[(c // GR) % 2, c % GR, 1]

        def gather_start(c):
            pltpu.async_copy(zs.at[src_ref(c)], rows.at[c % NB], sg[c % NB])

        def gather_wait(c):
            pltpu.make_async_copy(
                zs.at[src_ref(c)], rows.at[c % NB], sg[c % NB]).wait()

        def scale(c):
            b = c % NB

            def edge(e, carry):
                w = plsc.load_gather(
                    ex_v, [jnp.full((16,), c, _i32), jnp.full((16,), e, _i32)])
                for j in range(JW):
                    rows[b, e, pl.ds(j * 16, 16)] = (
                        rows[b, e, pl.ds(j * 16, 16)] * w)
                return carry
            lax.fori_loop(0, CH, edge, 0)

        def scat_start(c):
            pltpu.async_copy(
                rows.at[c % NB], acc_sh.at[dst_ref(c)], ss[c % NB], add=True)

        def scat_wait(c):
            pltpu.make_async_copy(
                rows.at[c % NB], acc_sh.at[dst_ref(c)], ss[c % NB]).wait()

        def stage2(c):
            gather_wait(c)
            if weighted:
                scale(c)
            scat_start(c)

        idx_start(0)
        plsc.subcore_barrier()
        for g in range(NG):
            idx_wait(g)
            for j in range(GR):
                c = g * GR + j
                if c >= NB:
                    scat_wait(c - NB)
                if j == min(NB, GR - 1) and g + 1 < NG:
                    idx_start(g + 1)
                gather_start(c)
                if c >= 1:
                    stage2(c - 1)
        stage2(RPT - 1)
        for c in range(RPT - NB, RPT):
            scat_wait(c)
        plsc.subcore_barrier()
        sl = pl.ds(sid * RSUB, RSUB)
        pltpu.sync_copy(acc_sh.at[sl], part.at[cid, sl])

    scratch = []
    if weighted:
        scratch.append(pltpu.VMEM((RPT, CH), _f32))
    scratch += [
        pltpu.VMEM((2, GR, 2, CH), _i32),
        pltpu.VMEM((NB, CH, F), _f32),
        pltpu.VMEM_SHARED((NPAD, F), _f32),
    ] + [pltpu.SemaphoreType.DMA] * (2 * NB + 2)

    return pl.kernel(
        body,
        mesh=_mesh,
        compiler_params=params,
        out_type=jax.ShapeDtypeStruct((NC, NPAD, F), _f32),
        scratch_types=scratch,
    )


_BN = 2048


def _make_encode(FI, H):
    def body(x_ref, w_ref, a1_ref, a2_ref, h_ref, s1_ref, s2_ref):
        h = jnp.dot(x_ref[...], w_ref[...], preferred_element_type=_f32)
        h_ref[...] = h
        s1_ref[...] = jnp.sum(h * a1_ref[...], axis=1, keepdims=True)
        s2_ref[...] = jnp.sum(h * a2_ref[...], axis=1, keepdims=True)

    return pl.pallas_call(
        body,
        grid=(NPAD // _BN,),
        in_specs=[
            pl.BlockSpec((_BN, FI), lambda i: (i, 0)),
            pl.BlockSpec((FI, H), lambda i: (0, 0)),
            pl.BlockSpec((1, H), lambda i: (0, 0)),
            pl.BlockSpec((1, H), lambda i: (0, 0)),
        ],
        out_specs=[
            pl.BlockSpec((_BN, H), lambda i: (i, 0)),
            pl.BlockSpec((_BN, 1), lambda i: (i, 0)),
            pl.BlockSpec((_BN, 1), lambda i: (i, 0)),
        ],
        out_shape=[
            jax.ShapeDtypeStruct((NPAD, H), _f32),
            jax.ShapeDtypeStruct((NPAD, 1), _f32),
            jax.ShapeDtypeStruct((NPAD, 1), _f32),
        ],
    )


def _make_h0fin(F):
    def body(p0, p1, den, dinv, h0_ref, zs_ref):
        h0 = (p0[...] + p1[...]) / (den[...] + 1e-16)
        h0_ref[...] = h0
        zs_ref[...] = h0 * dinv[...]

    return pl.pallas_call(
        body,
        grid=(NPAD // _BN,),
        in_specs=[
            pl.BlockSpec((_BN, F), lambda i: (i, 0)),
            pl.BlockSpec((_BN, F), lambda i: (i, 0)),
            pl.BlockSpec((_BN, 1), lambda i: (i, 0)),
            pl.BlockSpec((_BN, 1), lambda i: (i, 0)),
        ],
        out_specs=[
            pl.BlockSpec((_BN, F), lambda i: (i, 0)),
            pl.BlockSpec((_BN, F), lambda i: (i, 0)),
        ],
        out_shape=[
            jax.ShapeDtypeStruct((NPAD, F), _f32),
            jax.ShapeDtypeStruct((NPAD, F), _f32),
        ],
    )


def _make_fma2(F, relu):
    def body(p0, p1, rs, b, o_ref):
        v = rs[...] * (p0[...] + p1[...]) + ALPHA * b[...]
        if relu:
            v = jnp.maximum(v, 0.0)
        o_ref[...] = v

    return pl.pallas_call(
        body,
        grid=(NPAD // _BN,),
        in_specs=[
            pl.BlockSpec((_BN, F), lambda i: (i, 0)),
            pl.BlockSpec((_BN, F), lambda i: (i, 0)),
            pl.BlockSpec((_BN, 1), lambda i: (i, 0)),
            pl.BlockSpec((_BN, F), lambda i: (i, 0)),
        ],
        out_specs=pl.BlockSpec((_BN, F), lambda i: (i, 0)),
        out_shape=jax.ShapeDtypeStruct((NPAD, F), _f32),
    )


_enc1 = _make_encode(F_IN, HID)
_enc2 = _make_encode(HID, OUT)
_usp64 = _make_spmm(OUT, False, 8, 16, _sc_params_sp)
_wsp64 = _make_spmm(OUT, True, 4, 16, _sc_params_sp)
_usp128 = _make_spmm(HID, False, 2, 8, _sc_params)
_wsp128 = _make_spmm(HID, True, 2, 8, _sc_params)
_h0f64 = _make_h0fin(OUT)
_mid64 = _make_fma2(OUT, False)
_fin2 = _make_fma2(OUT, False)
_h0f128 = _make_h0fin(HID)
_mid128 = _make_fma2(HID, False)
_fin1 = _make_fma2(HID, True)


def _layer(halves, s1, s2, srcp, dstp, ep, zer, fin, dinv_in,
           _usp, _wsp, _h0f, _mid):
    ex, denp, degp = _attn(s1.reshape(NPAD), s2.reshape(NPAD), srcp, dstp)
    den = (denp[0] + denp[1]).reshape(NPAD, 1)
    if dinv_in is None:
        dinv = lax.rsqrt(degp[0] + degp[1] + 1.0)
    else:
        dinv = dinv_in
    dcol = dinv.reshape(NPAD, 1)
    c1 = ((1.0 - ALPHA) * dinv * dinv).reshape(NPAD, 1)
    rfin = ((1.0 - ALPHA) * dinv).reshape(NPAD, 1)
    h0s, zss, zs0s = [], [], []
    for hh in halves:
        p = _wsp(hh, ex, ep, zer)
        h0_i, zs_i = _h0f(p[0], p[1], den, dcol)
        h0s.append(h0_i)
        zs0s.append(zs_i)
        zss.append(zs_i)
    outs = [None] * len(halves)
    for t in range(K):
        for i in range(len(halves)):
            q = _usp(zss[i], ep, zer)
            if t < K - 1:
                zss[i] = _mid(q[0], q[1], c1, zs0s[i])
            else:
                outs[i] = fin(q[0], q[1], rfin, h0s[i])
    return outs, dinv


def kernel(x, edge_index, W1, a_src1, a_dst1, W2, a_src2, a_dst2):
    src = edge_index[0]
    dst = edge_index[1]
    pad = jnp.full((EPAD - E,), DUMMY, _i32)
    srcp = jnp.concatenate([src, pad]).reshape(RT, CH)
    dstp = jnp.concatenate([dst, pad]).reshape(RT, CH)
    ep = jnp.stack([srcp, dstp], axis=1)
    zer64 = jnp.zeros((RSUB, OUT), _f32)
    zer128 = jnp.zeros((RSUB, HID), _f32)
    xp = jnp.zeros((NPAD, F_IN), _f32).at[:N].set(x)
    h1, s11, s12 = _enc1(xp, W1, a_src1.reshape(1, HID), a_dst1.reshape(1, HID))
    outs1, dinv = _layer([h1], s11, s12, srcp, dstp, ep, zer128, _fin1, None,
                         _usp128, _wsp128, _h0f128, _mid128)
    x2 = outs1[0]
    h2, s21, s22 = _enc2(x2, W2, a_src2.reshape(1, OUT), a_dst2.reshape(1, OUT))
    outs2, _ = _layer([h2], s21, s22, srcp, dstp, ep, zer64, _fin2, dinv,
                      _usp64, _wsp64, _h0f64, _mid64)
    return outs2[0][:N]
